# baseline (speedup 1.0000x reference)
import jax
import jax.numpy as jnp
from jax.experimental import pallas as pl
from jax.experimental.pallas import tpu as pltpu

_N = 8192
_R = 64
_H = _N // 2


def _sort_body(x_ref, o_ref):
    z = x_ref[:]
    ia = jax.lax.broadcasted_iota(jnp.int32, (_H, 128), 0)
    il = jax.lax.broadcasted_iota(jnp.int32, (_H, 128), 1)
    ii = ia + jnp.where(il >= _R, _H, 0)
    k = 2
    while k <= _N:
        j = k // 2
        while j >= 1:
            is_lower = (ii & j) == 0
            if j < _H:
                fwd = pltpu.roll(z, _H - j, axis=0)
                bwd = pltpu.roll(z, j, axis=0)
                partner = jnp.where(is_lower, fwd, bwd)
            else:
                partner = pltpu.roll(z, _R, axis=1)
            up = (ii & k) == 0
            keep_min = is_lower == up
            z = jnp.where(keep_min, jnp.minimum(z, partner),
                          jnp.maximum(z, partner))
            j //= 2
        k *= 2
    o_ref[:] = z


def kernel(x):
    zt = x.reshape(_R, 2, _H).transpose(2, 1, 0).reshape(_H, 128)
    out = pl.pallas_call(
        _sort_body,
        out_shape=jax.ShapeDtypeStruct((_H, 128), jnp.float32),
    )(zt)
    return out.reshape(_H, 2, _R).transpose(2, 1, 0).reshape(_R, _N)


# speedup vs baseline: 1.5593x; 1.5593x over previous
import jax
import jax.numpy as jnp
from jax.experimental import pallas as pl
from jax.experimental.pallas import tpu as pltpu

_N = 8192
_R = 64
_H = _N // 2


def _sort_body(x_ref, o_ref):
    z = x_ref[:]
    ia = jax.lax.broadcasted_iota(jnp.int32, (_H, 128), 0)
    il = jax.lax.broadcasted_iota(jnp.int32, (_H, 128), 1)
    ii = ia + jnp.where(il >= _R, _H, 0)
    z = jnp.where((ii & 2) == 0, z, -z)
    k = 2
    while k <= _N:
        j = k // 2
        while j >= 1:
            is_lower = (ii & j) == 0
            if j < _H:
                fwd = pltpu.roll(z, _H - j, axis=0)
                bwd = pltpu.roll(z, j, axis=0)
            else:
                fwd = bwd = pltpu.roll(z, _R, axis=1)
            z = jnp.where(is_lower, jnp.minimum(z, fwd),
                          jnp.maximum(z, bwd))
            j //= 2
        if k < _N:
            z = jnp.where(((ii & k) != 0) == ((ii & (2 * k)) != 0), z, -z)
        k *= 2
    o_ref[:] = z


def kernel(x):
    zt = x.reshape(_R, 2, _H).transpose(2, 1, 0).reshape(_H, 128)
    out = pl.pallas_call(
        _sort_body,
        out_shape=jax.ShapeDtypeStruct((_H, 128), jnp.float32),
    )(zt)
    return out.reshape(_H, 2, _R).transpose(2, 1, 0).reshape(_R, _N)


# speedup vs baseline: 1.7816x; 1.1426x over previous
import jax
import jax.numpy as jnp
from jax.experimental import pallas as pl
from jax.experimental.pallas import tpu as pltpu

_N = 8192
_R = 64
_H = _N // 2


def _sort_body(x_ref, o_ref):
    z = x_ref[:]
    ia = jax.lax.broadcasted_iota(jnp.int32, (_H, 128), 0)
    il = jax.lax.broadcasted_iota(jnp.int32, (_H, 128), 1)
    ii = ia + jnp.where(il >= _R, _H, 0)
    z = jnp.where((ii & 2) == 0, z, -z)
    k = 2
    while k <= _N:
        j = k // 2
        while j >= 1:
            if j == _H:
                p = pltpu.roll(z, _R, axis=1)
                z = jnp.where((ii & j) == 0, jnp.minimum(z, p),
                              jnp.maximum(z, p))
            elif j >= 8:
                z4 = z.reshape(_H // (2 * j), 2, j, 128)
                a = z4[:, 0]
                b = z4[:, 1]
                z = jnp.concatenate(
                    [jnp.minimum(a, b)[:, None], jnp.maximum(a, b)[:, None]],
                    axis=1).reshape(_H, 128)
            else:
                fwd = pltpu.roll(z, _H - j, axis=0)
                bwd = pltpu.roll(z, j, axis=0)
                z = jnp.where((ii & j) == 0, jnp.minimum(z, fwd),
                              jnp.maximum(z, bwd))
            j //= 2
        if k < _N:
            z = jnp.where(((ii & k) != 0) == ((ii & (2 * k)) != 0), z, -z)
        k *= 2
    o_ref[:] = z


def kernel(x):
    zt = x.reshape(_R, 2, _H).transpose(2, 1, 0).reshape(_H, 128)
    out = pl.pallas_call(
        _sort_body,
        out_shape=jax.ShapeDtypeStruct((_H, 128), jnp.float32),
    )(zt)
    return out.reshape(_H, 2, _R).transpose(2, 1, 0).reshape(_R, _N)
